# Initial kernel scaffold; baseline (speedup 1.0000x reference)
import jax
import jax.numpy as jnp
from jax.experimental import pallas as pl


def kernel(pcl_pat, pcl_sample, params):
    raise NotImplementedError("write your pallas kernel here")


# speedup vs baseline: 4.6048x; 4.6048x over previous
import jax
import jax.numpy as jnp
from jax import lax
from jax.experimental import pallas as pl
from jax.experimental.pallas import tpu as pltpu

_N = 1024
_KNN = 16
_NPCL = 256


def _mm(x, w):
    return lax.dot_general(x, w, (((1,), (0,)), ((), ())),
                           preferred_element_type=jnp.float32,
                           precision=lax.Precision.HIGHEST)


def _mmb(x, w):
    return lax.dot_general(x.astype(jnp.bfloat16), w.astype(jnp.bfloat16),
                           (((1,), (0,)), ((), ())),
                           preferred_element_type=jnp.float32)


def _mm0(x, w):
    return lax.dot_general(x, w, (((0,), (0,)), ((), ())),
                           preferred_element_type=jnp.float32,
                           precision=lax.Precision.HIGHEST)


def _relu(x):
    return jnp.maximum(x, 0.0)


def _fusion_and_piles(pp, pos, code, num_pcl):
    m1w, m1b = pp['fusion']['mlp1'][0]
    m2w, m2b = pp['fusion']['mlp1'][1]
    h = _relu(_mmb(pos, m1w) + m1b)
    h = _mmb(h, m2w) + m2b
    L = pp['fusion']['lins']
    x = _relu(_mmb(h, L[0][0][:64]) + _mmb(code, L[0][0][64:]) + L[0][1])
    x = _relu(_mmb(x, L[1][0]) + L[1][1])
    x = _relu(_mmb(x, L[2][0]) + L[2][1])
    x = _relu(_mmb(x, L[3][0][:16]) + _mmb(h, L[3][0][16:80])
              + _mmb(code, L[3][0][80:]) + L[3][1])
    yy = _mmb(x, L[4][0]) + L[4][1]
    c1w, c1b = pp['conv_1']
    yc = _relu(_mmb(code, c1w[:48]) + _mmb(yy, c1w[48:]) + c1b)

    dist = jnp.sqrt(jnp.sum(pos * pos, axis=1, keepdims=True))
    dw = jax.nn.sigmoid(-pp['alpha'] * dist + pp['beta'])
    s = jnp.sum(dw, axis=0, keepdims=True)
    s = s + (s == 0.0).astype(jnp.float32) + 1e-6
    dw = dw / s * jnp.float32(_N)

    def pile(wb, x, npoint):
        w_, b_ = wb
        n, c = x.shape
        g = _mm0(dw[:n], x) * jnp.float32(1.0 / n)
        return _relu(_mmb(x[:npoint], w_[:c]) + _mmb(g, w_[c:]) + b_)

    y1 = pile(pp['pconv1'], yc, 2 * num_pcl)
    y2 = pile(pp['pconv2'], y1, num_pcl) + y1[:num_pcl]
    y3 = pile(pp['pconv3'], y2, num_pcl)
    y4 = pile(pp['pconv4'], y3, num_pcl // 2) + y3[:num_pcl // 2]
    y5 = pile(pp['pconv5'], y4, num_pcl // 2)
    c2w, c2b = pp['conv_2']
    c3w, c3b = pp['conv_3']
    out = _relu(_mmb(_relu(_mmb(y5, c2w) + c2b), c3w) + c3b)
    return out, dw


def _body(pat_ref, sam_ref, p, normal_ref, wgt_ref, nn_ref, d2_s):
    pv = jax.tree.map(lambda r: r[...], p)
    pos = pat_ref[0]

    sq = pos * pos
    q2 = _mm(sq, jnp.ones((3, 1), jnp.float32))
    p2 = lax.dot_general(jnp.ones((1, 3), jnp.float32), sq,
                         (((1,), (1,)), ((), ())),
                         preferred_element_type=jnp.float32,
                         precision=lax.Precision.HIGHEST)
    pos_bf = pos.astype(jnp.bfloat16)
    prod = lax.dot_general(pos_bf, pos_bf, (((1,), (1,)), ((), ())),
                           preferred_element_type=jnp.float32)
    d2_s[...] = q2 + p2 - 2.0 * prod

    li = lax.broadcasted_iota(jnp.int32, (_N, _N), 1)

    def extract(d2v):
        mv = jnp.min(d2v, axis=1, keepdims=True)
        am = jnp.min(jnp.where(d2v == mv, li, _N), axis=1, keepdims=True)
        return am

    d2v = d2_s[...]
    am0 = extract(d2v)
    d2_s[...] = jnp.where(li == am0, jnp.float32(jnp.inf), d2v)

    pe = pv['pe']
    w0, b0 = pe['enc'][0]
    ctb1 = _mmb(pos, w0[0:3])

    lane = lax.broadcasted_iota(jnp.int32, (1, 128), 1)

    def body1(k, carry):
        acc, idxf = carry
        d2v = d2_s[...]
        am = extract(d2v)
        ek = (lane == (k - 1)).astype(jnp.float32)
        idxf = idxf + am.astype(jnp.float32) * ek
        oh = (li == am).astype(jnp.float32)
        nb = _mm(oh, pos)
        acc = jnp.maximum(acc, _mmb(nb - pos, w0[3:6]))
        d2_s[...] = jnp.where(li == am, jnp.float32(jnp.inf), d2v)
        return acc, idxf

    acc1, idxf = lax.fori_loop(
        1, _KNN + 1, body1,
        (jnp.full((_N, 24), -jnp.inf, jnp.float32),
         jnp.zeros((_N, 128), jnp.float32)))
    feat1 = _relu(ctb1 + acc1 + b0)

    w1, b1 = pe['enc'][1]
    ctb2 = _mmb(feat1, w1[0:24])

    def body2(k, acc):
        ek = (lane == k).astype(jnp.float32)
        am = jnp.sum(idxf * ek, axis=1, keepdims=True).astype(jnp.int32)
        oh = (li == am).astype(jnp.float32)
        nb = _mm(oh, feat1)
        return jnp.maximum(acc, _mmb(nb - feat1, w1[24:48]))

    acc2 = lax.fori_loop(0, _KNN, body2,
                         jnp.full((_N, 48), -jnp.inf, jnp.float32))
    feat2 = _relu(ctb2 + acc2 + b1)

    out_pe, dw_pat = _fusion_and_piles(pe, pos, feat2, _NPCL)

    pos_s = sam_ref[0]
    pg = pv['pe_g']
    g0w, g0b = pg['enc'][0]
    g1w, g1b = pg['enc'][1]
    fg = _relu(_mmb(pos_s, g0w[0:3]) + g0b)
    fg = _relu(_mmb(fg, g1w[0:24]) + g1b)
    out_g, _ = _fusion_and_piles(pg, pos_s, fg, _NPCL)
    y_g = jnp.max(out_g, axis=0, keepdims=True)

    cpw, cpb = pv['conv_p']
    y0 = _relu(_mmb(out_pe, cpw[:128]) + _mmb(y_g, cpw[128:]) + cpb)
    wd = dw_pat[:128]

    def pileh(wb, x, npoint):
        w_, b_ = wb
        n, c = x.shape
        g = _mm0(wd[:n], x) * jnp.float32(1.0 / n)
        return _relu(_mmb(x[:npoint], w_[:c]) + _mmb(g, w_[c:]) + b_)

    y1 = pileh(pv['pconv_1'], y0, 128)
    y2 = pileh(pv['pconv_2'], y1, 64) + y1[:64] + y0[:64]
    feat = pileh(pv['pconv_3'], y2, 64)
    c1w, c1b = pv['conv_1']
    feat = _relu(_mmb(feat, c1w) + c1b)

    cww, cwb = pv['conv_w']
    wcol = 0.01 + jax.nn.sigmoid(_mmb(feat, cww) + cwb)
    wrow = 0.01 + jax.nn.sigmoid(
        lax.dot_general(cww.astype(jnp.bfloat16),
                        feat.astype(jnp.bfloat16),
                        (((0,), (1,)), ((), ())),
                        preferred_element_type=jnp.float32) + cwb)
    cnw, cnb = pv['conv_n']
    feat_w = _relu(_mmb(feat * wcol, cnw) + cnb)
    cqw, cqb = pv['conv_q']
    q = _mmb(feat_w, cqw) + cqb
    cvw, cvb = pv['conv_v']
    v = _mmb(feat_w, cvw) + cvb
    qm = jnp.max(q, axis=0, keepdims=True)
    e = jnp.exp(q - qm)
    sm = e / jnp.sum(e, axis=0, keepdims=True)
    attn = jnp.max(sm, axis=1, keepdims=True)
    fw = lax.dot_general(attn.astype(jnp.bfloat16),
                         v.astype(jnp.bfloat16),
                         (((0,), (0,)), ((), ())),
                         preferred_element_type=jnp.float32)
    n1w, n1b = pv['mlp_n1']
    fw = _mmb(fw, n1w) + n1b
    nw, nb_ = pv['mlp_n']
    n4 = _mmb(fw, nw) + nb_
    n3 = n4[:, 0:3]
    nrm = jnp.sqrt(jnp.sum(n3 * n3, axis=1, keepdims=True))
    normal_ref[0] = n3 / jnp.maximum(nrm, 1e-12)
    wgt_ref[0] = wrow
    nnw, nnb = pv['mlp_nn']
    nnf = _mmb(feat, nnw) + nnb
    nnrm = jnp.sqrt(jnp.sum(nnf * nnf, axis=1, keepdims=True))
    nn_ref[0] = nnf / jnp.maximum(nnrm, 1e-12)


def kernel(pcl_pat, pcl_sample, params):
    b = pcl_pat.shape[0]
    p2 = jax.tree.map(lambda a: a.reshape(1, -1) if a.ndim <= 1 else a,
                      params)
    w_specs = jax.tree.map(
        lambda a: pl.BlockSpec(a.shape, lambda i, _nd=a.ndim: (0,) * _nd), p2)
    normal, wrow, nn = pl.pallas_call(
        _body,
        grid=(b,),
        in_specs=[
            pl.BlockSpec((1, _N, 3), lambda i: (i, 0, 0)),
            pl.BlockSpec((1, _N, 3), lambda i: (i, 0, 0)),
            w_specs,
        ],
        out_specs=[
            pl.BlockSpec((1, 1, 3), lambda i: (i, 0, 0)),
            pl.BlockSpec((1, 1, 64), lambda i: (i, 0, 0)),
            pl.BlockSpec((1, 64, 3), lambda i: (i, 0, 0)),
        ],
        out_shape=[
            jax.ShapeDtypeStruct((b, 1, 3), jnp.float32),
            jax.ShapeDtypeStruct((b, 1, 64), jnp.float32),
            jax.ShapeDtypeStruct((b, 64, 3), jnp.float32),
        ],
        scratch_shapes=[
            pltpu.VMEM((_N, _N), jnp.float32),
        ],
        compiler_params=pltpu.CompilerParams(
            dimension_semantics=("arbitrary",)),
    )(pcl_pat, pcl_sample, p2)
    return (normal.reshape(b, 3), wrow, nn)


# speedup vs baseline: 10.1401x; 2.2021x over previous
import jax
import jax.numpy as jnp
from jax import lax
from jax.experimental import pallas as pl
from jax.experimental.pallas import tpu as pltpu

_N = 1024
_KNN = 16
_NPCL = 256


def _mm(x, w):
    return lax.dot_general(x, w, (((1,), (0,)), ((), ())),
                           preferred_element_type=jnp.float32,
                           precision=lax.Precision.HIGHEST)


def _mmb(x, w):
    return lax.dot_general(x.astype(jnp.bfloat16), w.astype(jnp.bfloat16),
                           (((1,), (0,)), ((), ())),
                           preferred_element_type=jnp.float32)


def _mm0(x, w):
    return lax.dot_general(x, w, (((0,), (0,)), ((), ())),
                           preferred_element_type=jnp.float32,
                           precision=lax.Precision.HIGHEST)


def _relu(x):
    return jnp.maximum(x, 0.0)


def _split_bf(x):
    hi = x.astype(jnp.bfloat16)
    lo = (x - hi.astype(jnp.float32)).astype(jnp.bfloat16)
    return jnp.concatenate([hi, lo], axis=1)


def _gather(ohb, hilo, c):
    d = (((1,), (0,)), ((), ()))
    nbh = lax.dot_general(ohb, hilo, d, preferred_element_type=jnp.float32)
    return nbh[:, :c] + nbh[:, c:]


def _fusion_and_piles(pp, pos, code, num_pcl):
    m1w, m1b = pp['fusion']['mlp1'][0]
    m2w, m2b = pp['fusion']['mlp1'][1]
    h = _relu(_mmb(pos, m1w) + m1b)
    h = _mmb(h, m2w) + m2b
    L = pp['fusion']['lins']
    x = _relu(_mmb(h, L[0][0][:64]) + _mmb(code, L[0][0][64:]) + L[0][1])
    x = _relu(_mmb(x, L[1][0]) + L[1][1])
    x = _relu(_mmb(x, L[2][0]) + L[2][1])
    x = _relu(_mmb(x, L[3][0][:16]) + _mmb(h, L[3][0][16:80])
              + _mmb(code, L[3][0][80:]) + L[3][1])
    yy = _mmb(x, L[4][0]) + L[4][1]
    c1w, c1b = pp['conv_1']
    yc = _relu(_mmb(code, c1w[:48]) + _mmb(yy, c1w[48:]) + c1b)

    dist = jnp.sqrt(jnp.sum(pos * pos, axis=1, keepdims=True))
    dw = jax.nn.sigmoid(-pp['alpha'] * dist + pp['beta'])
    s = jnp.sum(dw, axis=0, keepdims=True)
    s = s + (s == 0.0).astype(jnp.float32) + 1e-6
    dw = dw / s * jnp.float32(_N)

    def pile(wb, x, npoint):
        w_, b_ = wb
        n, c = x.shape
        g = jnp.sum(x * dw[:n], axis=0, keepdims=True) * jnp.float32(1.0 / n)
        return _relu(_mmb(x[:npoint], w_[:c]) + _mmb(g, w_[c:]) + b_)

    y1 = pile(pp['pconv1'], yc, 2 * num_pcl)
    y2 = pile(pp['pconv2'], y1, num_pcl) + y1[:num_pcl]
    y3 = pile(pp['pconv3'], y2, num_pcl)
    y4 = pile(pp['pconv4'], y3, num_pcl // 2) + y3[:num_pcl // 2]
    y5 = pile(pp['pconv5'], y4, num_pcl // 2)
    c2w, c2b = pp['conv_2']
    c3w, c3b = pp['conv_3']
    out = _relu(_mmb(_relu(_mmb(y5, c2w) + c2b), c3w) + c3b)
    return out, dw


def _body(pat_ref, sam_ref, p, normal_ref, wgt_ref, nn_ref, d2_s):
    pv = jax.tree.map(lambda r: r[...], p)
    pos = pat_ref[0]

    sq = pos * pos
    q2 = _mm(sq, jnp.ones((3, 1), jnp.float32))
    p2 = lax.dot_general(jnp.ones((1, 3), jnp.float32), sq,
                         (((1,), (1,)), ((), ())),
                         preferred_element_type=jnp.float32,
                         precision=lax.Precision.HIGHEST)
    pos_bf = pos.astype(jnp.bfloat16)
    prod = lax.dot_general(pos_bf, pos_bf, (((1,), (1,)), ((), ())),
                           preferred_element_type=jnp.float32)
    d2_s[...] = q2 + p2 - 2.0 * prod

    li = lax.broadcasted_iota(jnp.int32, (_N, _N), 1)

    d2v = d2_s[...]
    mv0 = jnp.min(d2v, axis=1, keepdims=True)
    am0 = jnp.min(jnp.where(d2v == mv0, li, _N), axis=1, keepdims=True)

    pe = pv['pe']
    w0, b0 = pe['enc'][0]
    ctb1 = _mmb(pos, w0[0:3])
    pos_g = _split_bf(pos)

    lane = lax.broadcasted_iota(jnp.int32, (1, 128), 1)

    def body1(k, carry):
        acc, idxf, pv, pi = carry
        d2v = d2_s[...]
        beat = (d2v > pv) | ((d2v == pv) & (li > pi))
        cand = jnp.where(beat, d2v, jnp.float32(jnp.inf))
        mv = jnp.min(cand, axis=1, keepdims=True)
        am = jnp.min(jnp.where(cand == mv, li, _N), axis=1, keepdims=True)
        ek = (lane == (k - 1)).astype(jnp.float32)
        idxf = idxf + am.astype(jnp.float32) * ek
        ohb = (li == am).astype(jnp.bfloat16)
        nb = _gather(ohb, pos_g, 3)
        acc = jnp.maximum(acc, _mmb(nb - pos, w0[3:6]))
        return acc, idxf, mv, am

    acc1, idxf, _, _ = lax.fori_loop(
        1, _KNN + 1, body1,
        (jnp.full((_N, 24), -jnp.inf, jnp.float32),
         jnp.zeros((_N, 128), jnp.float32), mv0, am0))
    feat1 = _relu(ctb1 + acc1 + b0)

    w1, b1 = pe['enc'][1]
    ctb2 = _mmb(feat1, w1[0:24])
    f1_g = _split_bf(feat1)

    def body2(k, acc):
        ek = (lane == k).astype(jnp.float32)
        am = jnp.sum(idxf * ek, axis=1, keepdims=True).astype(jnp.int32)
        ohb = (li == am).astype(jnp.bfloat16)
        nb = _gather(ohb, f1_g, 24)
        return jnp.maximum(acc, _mmb(nb - feat1, w1[24:48]))

    acc2 = lax.fori_loop(0, _KNN, body2,
                         jnp.full((_N, 48), -jnp.inf, jnp.float32))
    feat2 = _relu(ctb2 + acc2 + b1)

    out_pe, dw_pat = _fusion_and_piles(pe, pos, feat2, _NPCL)

    pos_s = sam_ref[0]
    pg = pv['pe_g']
    g0w, g0b = pg['enc'][0]
    g1w, g1b = pg['enc'][1]
    fg = _relu(_mmb(pos_s, g0w[0:3]) + g0b)
    fg = _relu(_mmb(fg, g1w[0:24]) + g1b)
    out_g, _ = _fusion_and_piles(pg, pos_s, fg, _NPCL)
    y_g = jnp.max(out_g, axis=0, keepdims=True)

    cpw, cpb = pv['conv_p']
    y0 = _relu(_mmb(out_pe, cpw[:128]) + _mmb(y_g, cpw[128:]) + cpb)
    wd = dw_pat[:128]

    def pileh(wb, x, npoint):
        w_, b_ = wb
        n, c = x.shape
        g = jnp.sum(x * wd[:n], axis=0, keepdims=True) * jnp.float32(1.0 / n)
        return _relu(_mmb(x[:npoint], w_[:c]) + _mmb(g, w_[c:]) + b_)

    y1 = pileh(pv['pconv_1'], y0, 128)
    y2 = pileh(pv['pconv_2'], y1, 64) + y1[:64] + y0[:64]
    feat = pileh(pv['pconv_3'], y2, 64)
    c1w, c1b = pv['conv_1']
    feat = _relu(_mmb(feat, c1w) + c1b)

    cww, cwb = pv['conv_w']
    wcol = 0.01 + jax.nn.sigmoid(_mmb(feat, cww) + cwb)
    wrow = 0.01 + jax.nn.sigmoid(
        lax.dot_general(cww.astype(jnp.bfloat16),
                        feat.astype(jnp.bfloat16),
                        (((0,), (1,)), ((), ())),
                        preferred_element_type=jnp.float32) + cwb)
    cnw, cnb = pv['conv_n']
    feat_w = _relu(_mmb(feat * wcol, cnw) + cnb)
    cqw, cqb = pv['conv_q']
    q = _mmb(feat_w, cqw) + cqb
    cvw, cvb = pv['conv_v']
    v = _mmb(feat_w, cvw) + cvb
    qm = jnp.max(q, axis=0, keepdims=True)
    e = jnp.exp(q - qm)
    sm = e / jnp.sum(e, axis=0, keepdims=True)
    attn = jnp.max(sm, axis=1, keepdims=True)
    fw = lax.dot_general(attn.astype(jnp.bfloat16),
                         v.astype(jnp.bfloat16),
                         (((0,), (0,)), ((), ())),
                         preferred_element_type=jnp.float32)
    n1w, n1b = pv['mlp_n1']
    fw = _mmb(fw, n1w) + n1b
    nw, nb_ = pv['mlp_n']
    n4 = _mmb(fw, nw) + nb_
    n3 = n4[:, 0:3]
    nrm = jnp.sqrt(jnp.sum(n3 * n3, axis=1, keepdims=True))
    normal_ref[0] = n3 / jnp.maximum(nrm, 1e-12)
    wgt_ref[0] = wrow
    nnw, nnb = pv['mlp_nn']
    nnf = _mmb(feat, nnw) + nnb
    nnrm = jnp.sqrt(jnp.sum(nnf * nnf, axis=1, keepdims=True))
    nn_ref[0] = nnf / jnp.maximum(nnrm, 1e-12)


def kernel(pcl_pat, pcl_sample, params):
    b = pcl_pat.shape[0]
    p2 = jax.tree.map(lambda a: a.reshape(1, -1) if a.ndim <= 1 else a,
                      params)
    w_specs = jax.tree.map(
        lambda a: pl.BlockSpec(a.shape, lambda i, _nd=a.ndim: (0,) * _nd), p2)
    normal, wrow, nn = pl.pallas_call(
        _body,
        grid=(b,),
        in_specs=[
            pl.BlockSpec((1, _N, 3), lambda i: (i, 0, 0)),
            pl.BlockSpec((1, _N, 3), lambda i: (i, 0, 0)),
            w_specs,
        ],
        out_specs=[
            pl.BlockSpec((1, 1, 3), lambda i: (i, 0, 0)),
            pl.BlockSpec((1, 1, 64), lambda i: (i, 0, 0)),
            pl.BlockSpec((1, 64, 3), lambda i: (i, 0, 0)),
        ],
        out_shape=[
            jax.ShapeDtypeStruct((b, 1, 3), jnp.float32),
            jax.ShapeDtypeStruct((b, 1, 64), jnp.float32),
            jax.ShapeDtypeStruct((b, 64, 3), jnp.float32),
        ],
        scratch_shapes=[
            pltpu.VMEM((_N, _N), jnp.float32),
        ],
        compiler_params=pltpu.CompilerParams(
            dimension_semantics=("parallel",)),
    )(pcl_pat, pcl_sample, p2)
    return (normal.reshape(b, 3), wrow, nn)
